# baseline (speedup 1.0000x reference)
import jax
import jax.numpy as jnp
from jax import lax
from jax.experimental import pallas as pl
from jax.experimental.pallas import tpu as pltpu

_VARIANCE = (0.1, 0.2)
_THRESHOLD = 0.35
_NEG_POS_RATIO = 3


def _prior_setup(pr_ref, pp_scr, cs_scr, it_scr):
    T = it_scr.shape[0]
    P = pr_ref.shape[1]
    v0, _ = _VARIANCE
    pcx = pr_ref[0:1, :]
    pcy = pr_ref[1:2, :]
    pw = pr_ref[2:3, :]
    ph = pr_ref[3:4, :]
    px1 = pcx - pw / 2
    py1 = pcy - ph / 2
    px2 = pcx + pw / 2
    py2 = pcy + ph / 2
    area_p = (px2 - px1) * (py2 - py1)
    lane8 = lax.broadcasted_iota(jnp.int32, (8, P), 0)
    pp = jnp.where(lane8 == 0, jnp.broadcast_to(px1, (8, P)),
                   jnp.where(lane8 == 1, jnp.broadcast_to(py1, (8, P)),
                             jnp.where(lane8 == 2, jnp.broadcast_to(px2, (8, P)),
                                       jnp.where(lane8 == 3, jnp.broadcast_to(py2, (8, P)),
                                                 jnp.broadcast_to(area_p, (8, P))))))
    pp_scr[...] = pp
    sub = lax.broadcasted_iota(jnp.int32, (16, P), 0)
    is_y = (sub & 1) == 1
    is_log = jnp.logical_or(sub == 2, sub == 3)
    pcx16 = jnp.broadcast_to(pcx, (16, P))
    pcy16 = jnp.broadcast_to(pcy, (16, P))
    pw16 = jnp.broadcast_to(pw, (16, P))
    ph16 = jnp.broadcast_to(ph, (16, P))
    center = jnp.where(is_log, 0.0, jnp.where(is_y, pcy16, pcx16))
    scale = jnp.where(is_log, 1.0, v0) * jnp.where(is_y, ph16, pw16)
    cs_scr[0:16] = center
    cs_scr[16:32] = scale
    it_scr[...] = lax.broadcasted_iota(jnp.int32, (T, P), 0)


def _image_step(i, tgt_ref, pr_ref, loc_ref, conf_ref, m_scr, st_scr,
                pp_scr, cs_scr, it_scr):
    T = tgt_ref.shape[1]
    P = pr_ref.shape[1]
    f32 = jnp.float32

    px1 = pp_scr[0:1, :]
    py1 = pp_scr[1:2, :]
    px2 = pp_scr[2:3, :]
    py2 = pp_scr[3:4, :]
    area_p = pp_scr[4:5, :]

    tgt = tgt_ref[0]
    tx1 = tgt[:, 0:1]
    ty1 = tgt[:, 1:2]
    tx2 = tgt[:, 2:3]
    ty2 = tgt[:, 3:4]
    area_t = (tx2 - tx1) * (ty2 - ty1)

    iw = jnp.clip(jnp.minimum(px2, tx2) - jnp.maximum(px1, tx1), 0.0, None)
    ih = jnp.clip(jnp.minimum(py2, ty2) - jnp.maximum(py1, ty1), 0.0, None)
    inter = iw * ih
    union = area_t + area_p - inter
    iou = inter / union

    iota_t = it_scr[...]
    iota_p = lax.broadcasted_iota(jnp.int32, (1, P), 1)

    best_ov = jnp.max(iou, axis=0, keepdims=True)
    bti = jnp.argmax(iou, axis=0, keepdims=True)

    bpi = jnp.argmax(iou, axis=1, keepdims=True)
    bestmask = jnp.any(iota_p == bpi, axis=0, keepdims=True)

    pos = jnp.logical_or(best_ov >= _THRESHOLD, bestmask)
    posf = pos.astype(f32)
    num_pos = jnp.sum(posf)

    oh = (bti == iota_t).astype(f32)
    u0 = (tgt[:, 0:1] + tgt[:, 2:3]) / 2
    u1 = (tgt[:, 1:2] + tgt[:, 3:4]) / 2
    u2 = tgt[:, 2:3] - tgt[:, 0:1]
    u3 = tgt[:, 3:4] - tgt[:, 1:2]
    tgt_u = jnp.concatenate([u0, u1, u2, u3, tgt[:, 8:20]], axis=1)
    u = lax.dot_general(tgt_u, oh, (((0,), (0,)), ((), ())),
                        preferred_element_type=f32)

    _, v1 = _VARIANCE
    sub = lax.broadcasted_iota(jnp.int32, (16, P), 0)
    is_log = jnp.logical_or(sub == 2, sub == 3)
    center = cs_scr[0:16]
    scale = cs_scr[16:32]
    w = (u - center) / scale
    g = jnp.where(is_log, jnp.log(w) / v1, w)

    loc = loc_ref[0]
    d = loc - g
    ad = jnp.abs(d)
    sl1 = jnp.where(ad < 1.0, 0.5 * d * d, ad - 0.5)
    loss_l_img = jnp.sum(sl1 * posf)

    x = conf_ref[0]
    lca = jnp.maximum(x, 0.0) - x * posf + jnp.log1p(jnp.exp(-jnp.abs(x)))
    posbce = jnp.sum(lca * posf)

    m_scr[pl.ds(i, 1), :] = jnp.where(pos, 0.0, lca)
    lane = lax.broadcasted_iota(jnp.int32, (1, 8), 1)
    row = jnp.where(lane == 0, num_pos,
                    jnp.where(lane == 1, loss_l_img,
                              jnp.where(lane == 2, posbce, 0.0)))
    st_scr[pl.ds(i, 1), :] = row


def _final_step(out_ref, m_scr, st_scr):
    P = m_scr.shape[1]
    m = m_scr[...]
    bits = lax.bitcast_convert_type(m, jnp.int32)
    npos = st_scr[:, 0:1]
    k = jnp.minimum(_NEG_POS_RATIO * npos, float(P - 1))

    hi = jnp.right_shift(bits, 16).astype(jnp.int16)

    def body_hi(i, prefix):
        bit = 14 - i
        trial = prefix | jnp.left_shift(jnp.int32(1), bit)
        cnt = jnp.sum((hi >= trial.astype(jnp.int16)).astype(jnp.float32),
                      axis=1, keepdims=True)
        return jnp.where(cnt >= k, trial, prefix)

    p_hi0 = jnp.zeros(npos.shape, jnp.int32)
    p_hi32 = lax.fori_loop(0, 15, body_hi, p_hi0)
    p_hi = p_hi32.astype(jnp.int16)

    c_above = jnp.sum((hi > p_hi).astype(jnp.float32), axis=1, keepdims=True)
    lo_u = jnp.bitwise_and(bits, 0xFFFF)
    lo_s = (lo_u - 32768).astype(jnp.int16)
    lo_m = jnp.where(hi == p_hi, lo_s, jnp.int16(-32768))

    def body_lo(i, prefix):
        bit = 15 - i
        trial = prefix | jnp.left_shift(jnp.int32(1), bit)
        trial_s = (trial - 32768).astype(jnp.int16)
        cnt = c_above + jnp.sum((lo_m >= trial_s).astype(jnp.float32),
                                axis=1, keepdims=True)
        return jnp.where(cnt >= k, trial, prefix)

    p_lo0 = jnp.zeros(npos.shape, jnp.int32)
    p_lo = lax.fori_loop(0, 16, body_lo, p_lo0)

    prefix = jnp.left_shift(p_hi32, 16) | p_lo
    tval = lax.bitcast_convert_type(prefix, jnp.float32)
    gt = (m > tval).astype(jnp.float32)
    cgt = jnp.sum(gt, axis=1, keepdims=True)
    sgt = jnp.sum(m * gt, axis=1, keepdims=True)
    negsum = jnp.where(k > 0.0, sgt + (k - cgt) * tval, 0.0)

    loss_l = jnp.sum(st_scr[:, 1:2])
    loss_c = jnp.sum(st_scr[:, 2:3] + negsum)
    n = jnp.sum(npos)
    lane = lax.broadcasted_iota(jnp.int32, (1, 2), 1)
    out_ref[...] = jnp.where(lane == 0, loss_l / n, loss_c / n)


def _make_body(num):
    def body(tgt_ref, pr_ref, loc_ref, conf_ref, out_ref, m_scr, st_scr,
             pp_scr, cs_scr, it_scr):
        i = pl.program_id(0)

        @pl.when(i == 0)
        def _():
            _prior_setup(pr_ref, pp_scr, cs_scr, it_scr)

        _image_step(i, tgt_ref, pr_ref, loc_ref, conf_ref, m_scr, st_scr,
                    pp_scr, cs_scr, it_scr)

        @pl.when(i == num - 1)
        def _():
            _final_step(out_ref, m_scr, st_scr)

    return body


def _build(num, P, T):
    return pl.pallas_call(
        _make_body(num),
        grid=(num,),
        in_specs=[
            pl.BlockSpec((1, T, 20), lambda i: (i, 0, 0)),
            pl.BlockSpec((4, P), lambda i: (0, 0)),
            pl.BlockSpec((1, 16, P), lambda i: (i, 0, 0)),
            pl.BlockSpec((1, 1, P), lambda i: (i, 0, 0)),
        ],
        out_specs=pl.BlockSpec((1, 2), lambda i: (0, 0)),
        out_shape=jax.ShapeDtypeStruct((1, 2), jnp.float32),
        scratch_shapes=[
            pltpu.VMEM((num, P), jnp.float32),
            pltpu.VMEM((num, 8), jnp.float32),
            pltpu.VMEM((8, P), jnp.float32),
            pltpu.VMEM((32, P), jnp.float32),
            pltpu.VMEM((T, P), jnp.int32),
        ],
    )


def kernel(loc_data, conf_data, targets, priors):
    num, P = loc_data.shape[0], loc_data.shape[1]
    T = targets.shape[1]
    loc_t = jnp.transpose(loc_data, (0, 2, 1))
    conf = jnp.transpose(conf_data, (0, 2, 1))
    pr_t = priors[:P, :].T
    out = _build(num, P, T)(targets, pr_t, loc_t, conf)
    return out[0, 0], out[0, 1]


# speedup vs baseline: 1.0486x; 1.0486x over previous
import jax
import jax.numpy as jnp
from jax import lax
from jax.experimental import pallas as pl
from jax.experimental.pallas import tpu as pltpu

_VARIANCE = (0.1, 0.2)
_THRESHOLD = 0.35
_NEG_POS_RATIO = 3


def _prior_setup(pr_ref, pp_scr, cs_scr, it_scr):
    T = it_scr.shape[0]
    P = pr_ref.shape[1]
    v0, _ = _VARIANCE
    pcx = pr_ref[0:1, :]
    pcy = pr_ref[1:2, :]
    pw = pr_ref[2:3, :]
    ph = pr_ref[3:4, :]
    px1 = pcx - pw / 2
    py1 = pcy - ph / 2
    px2 = pcx + pw / 2
    py2 = pcy + ph / 2
    area_p = (px2 - px1) * (py2 - py1)
    lane8 = lax.broadcasted_iota(jnp.int32, (8, P), 0)
    pp = jnp.where(lane8 == 0, jnp.broadcast_to(px1, (8, P)),
                   jnp.where(lane8 == 1, jnp.broadcast_to(py1, (8, P)),
                             jnp.where(lane8 == 2, jnp.broadcast_to(px2, (8, P)),
                                       jnp.where(lane8 == 3, jnp.broadcast_to(py2, (8, P)),
                                                 jnp.broadcast_to(area_p, (8, P))))))
    pp_scr[...] = pp
    sub = lax.broadcasted_iota(jnp.int32, (16, P), 0)
    is_y = (sub & 1) == 1
    is_log = jnp.logical_or(sub == 2, sub == 3)
    pcx16 = jnp.broadcast_to(pcx, (16, P))
    pcy16 = jnp.broadcast_to(pcy, (16, P))
    pw16 = jnp.broadcast_to(pw, (16, P))
    ph16 = jnp.broadcast_to(ph, (16, P))
    center = jnp.where(is_log, 0.0, jnp.where(is_y, pcy16, pcx16))
    scale = jnp.where(is_log, 1.0, v0) * jnp.where(is_y, ph16, pw16)
    cs_scr[0:16] = center
    cs_scr[16:32] = scale
    it_scr[...] = lax.broadcasted_iota(jnp.int32, (T, P), 0)


def _image_step(i, tgt_ref, pr_ref, loc_ref, conf_ref, m_scr, st_scr,
                pp_scr, cs_scr, it_scr):
    T = tgt_ref.shape[1]
    P = pr_ref.shape[1]
    f32 = jnp.float32

    px1 = pp_scr[0:1, :]
    py1 = pp_scr[1:2, :]
    px2 = pp_scr[2:3, :]
    py2 = pp_scr[3:4, :]
    area_p = pp_scr[4:5, :]

    tgt = tgt_ref[0]
    tx1 = tgt[:, 0:1]
    ty1 = tgt[:, 1:2]
    tx2 = tgt[:, 2:3]
    ty2 = tgt[:, 3:4]
    area_t = (tx2 - tx1) * (ty2 - ty1)

    iw = jnp.clip(jnp.minimum(px2, tx2) - jnp.maximum(px1, tx1), 0.0, None)
    ih = jnp.clip(jnp.minimum(py2, ty2) - jnp.maximum(py1, ty1), 0.0, None)
    inter = iw * ih
    union = area_t + area_p - inter
    iou = inter / union

    iota_t = it_scr[...]
    iota_p = lax.broadcasted_iota(jnp.int32, (1, P), 1)

    best_ov = jnp.max(iou, axis=0, keepdims=True)
    bti = jnp.argmax(iou, axis=0, keepdims=True)

    bpi = jnp.argmax(iou, axis=1, keepdims=True)
    bestmask = jnp.any(iota_p == bpi, axis=0, keepdims=True)

    pos = jnp.logical_or(best_ov >= _THRESHOLD, bestmask)
    posf = pos.astype(f32)
    num_pos = jnp.sum(posf)

    oh = (bti == iota_t).astype(f32)
    u0 = (tgt[:, 0:1] + tgt[:, 2:3]) / 2
    u1 = (tgt[:, 1:2] + tgt[:, 3:4]) / 2
    u2 = tgt[:, 2:3] - tgt[:, 0:1]
    u3 = tgt[:, 3:4] - tgt[:, 1:2]
    tgt_u = jnp.concatenate([u0, u1, u2, u3, tgt[:, 8:20]], axis=1)
    u = lax.dot_general(tgt_u, oh, (((0,), (0,)), ((), ())),
                        preferred_element_type=f32)

    _, v1 = _VARIANCE
    sub = lax.broadcasted_iota(jnp.int32, (16, P), 0)
    is_log = jnp.logical_or(sub == 2, sub == 3)
    center = cs_scr[0:16]
    scale = cs_scr[16:32]
    w = (u - center) / scale
    g = jnp.where(is_log, jnp.log(w) / v1, w)

    loc = loc_ref[0]
    d = loc - g
    ad = jnp.abs(d)
    sl1 = jnp.where(ad < 1.0, 0.5 * d * d, ad - 0.5)
    loss_l_img = jnp.sum(sl1 * posf)

    x = conf_ref[0]
    lca = jnp.maximum(x, 0.0) - x * posf + jnp.log1p(jnp.exp(-jnp.abs(x)))
    posbce = jnp.sum(lca * posf)

    m_scr[pl.ds(i, 1), :] = jnp.where(pos, 0.0, lca)
    lane = lax.broadcasted_iota(jnp.int32, (1, 8), 1)
    row = jnp.where(lane == 0, num_pos,
                    jnp.where(lane == 1, loss_l_img,
                              jnp.where(lane == 2, posbce, 0.0)))
    st_scr[pl.ds(i, 1), :] = row


def _final_step(out_ref, m_scr, st_scr):
    P = m_scr.shape[1]
    m = m_scr[...]
    bits = lax.bitcast_convert_type(m, jnp.int32)
    npos = st_scr[:, 0:1]
    k = jnp.minimum(_NEG_POS_RATIO * npos, float(P - 1))

    def body(i, prefix):
        bit = 30 - i
        trial = prefix | jnp.left_shift(jnp.int32(1), bit)
        cnt = jnp.sum((bits >= trial).astype(jnp.float32), axis=1,
                      keepdims=True)
        return jnp.where(cnt >= k, trial, prefix)

    prefix0 = jnp.zeros(npos.shape, jnp.int32)
    prefix = lax.fori_loop(0, 31, body, prefix0)
    tval = lax.bitcast_convert_type(prefix, jnp.float32)
    gt = (m > tval).astype(jnp.float32)
    cgt = jnp.sum(gt, axis=1, keepdims=True)
    sgt = jnp.sum(m * gt, axis=1, keepdims=True)
    negsum = jnp.where(k > 0.0, sgt + (k - cgt) * tval, 0.0)

    loss_l = jnp.sum(st_scr[:, 1:2])
    loss_c = jnp.sum(st_scr[:, 2:3] + negsum)
    n = jnp.sum(npos)
    lane = lax.broadcasted_iota(jnp.int32, (1, 2), 1)
    out_ref[...] = jnp.where(lane == 0, loss_l / n, loss_c / n)


def _make_body(num):
    def body(tgt_ref, pr_ref, loc_ref, conf_ref, out_ref, m_scr, st_scr,
             pp_scr, cs_scr, it_scr):
        i = pl.program_id(0)

        @pl.when(i == 0)
        def _():
            _prior_setup(pr_ref, pp_scr, cs_scr, it_scr)

        _image_step(i, tgt_ref, pr_ref, loc_ref, conf_ref, m_scr, st_scr,
                    pp_scr, cs_scr, it_scr)

        @pl.when(i == num - 1)
        def _():
            _final_step(out_ref, m_scr, st_scr)

    return body


def _build(num, P, T):
    return pl.pallas_call(
        _make_body(num),
        grid=(num,),
        in_specs=[
            pl.BlockSpec((1, T, 20), lambda i: (i, 0, 0)),
            pl.BlockSpec((4, P), lambda i: (0, 0)),
            pl.BlockSpec((1, 16, P), lambda i: (i, 0, 0)),
            pl.BlockSpec((1, 1, P), lambda i: (i, 0, 0)),
        ],
        out_specs=pl.BlockSpec((1, 2), lambda i: (0, 0)),
        out_shape=jax.ShapeDtypeStruct((1, 2), jnp.float32),
        scratch_shapes=[
            pltpu.VMEM((num, P), jnp.float32),
            pltpu.VMEM((num, 8), jnp.float32),
            pltpu.VMEM((8, P), jnp.float32),
            pltpu.VMEM((32, P), jnp.float32),
            pltpu.VMEM((T, P), jnp.int32),
        ],
    )


def kernel(loc_data, conf_data, targets, priors):
    num, P = loc_data.shape[0], loc_data.shape[1]
    T = targets.shape[1]
    loc_t = jnp.transpose(loc_data, (0, 2, 1))
    conf = jnp.transpose(conf_data, (0, 2, 1))
    pr_t = priors[:P, :].T
    out = _build(num, P, T)(targets, pr_t, loc_t, conf)
    return out[0, 0], out[0, 1]


# speedup vs baseline: 1.0571x; 1.0082x over previous
import jax
import jax.numpy as jnp
from jax import lax
from jax.experimental import pallas as pl
from jax.experimental.pallas import tpu as pltpu

_VARIANCE = (0.1, 0.2)
_THRESHOLD = 0.35
_NEG_POS_RATIO = 3


def _prior_setup(pr_ref, pp_scr, cs_scr, it_scr):
    T = it_scr.shape[0]
    P = pr_ref.shape[1]
    v0, _ = _VARIANCE
    pcx = pr_ref[0:1, :]
    pcy = pr_ref[1:2, :]
    pw = pr_ref[2:3, :]
    ph = pr_ref[3:4, :]
    px1 = pcx - pw / 2
    py1 = pcy - ph / 2
    px2 = pcx + pw / 2
    py2 = pcy + ph / 2
    area_p = (px2 - px1) * (py2 - py1)
    lane8 = lax.broadcasted_iota(jnp.int32, (8, P), 0)
    pp = jnp.where(lane8 == 0, jnp.broadcast_to(px1, (8, P)),
                   jnp.where(lane8 == 1, jnp.broadcast_to(py1, (8, P)),
                             jnp.where(lane8 == 2, jnp.broadcast_to(px2, (8, P)),
                                       jnp.where(lane8 == 3, jnp.broadcast_to(py2, (8, P)),
                                                 jnp.broadcast_to(area_p, (8, P))))))
    pp_scr[...] = pp
    sub = lax.broadcasted_iota(jnp.int32, (16, P), 0)
    is_y = (sub & 1) == 1
    is_log = jnp.logical_or(sub == 2, sub == 3)
    pcx16 = jnp.broadcast_to(pcx, (16, P))
    pcy16 = jnp.broadcast_to(pcy, (16, P))
    pw16 = jnp.broadcast_to(pw, (16, P))
    ph16 = jnp.broadcast_to(ph, (16, P))
    center = jnp.where(is_log, 0.0, jnp.where(is_y, pcy16, pcx16))
    scale = jnp.where(is_log, 1.0, v0) * jnp.where(is_y, ph16, pw16)
    cs_scr[0:16] = center
    cs_scr[16:32] = 1.0 / scale
    it_scr[...] = lax.broadcasted_iota(jnp.int32, (T, P), 0)


def _image_step(i, tgt_ref, pr_ref, loc_ref, conf_ref, m_scr, st_scr,
                pp_scr, cs_scr, it_scr):
    T = tgt_ref.shape[1]
    P = pr_ref.shape[1]
    f32 = jnp.float32

    px1 = pp_scr[0:1, :]
    py1 = pp_scr[1:2, :]
    px2 = pp_scr[2:3, :]
    py2 = pp_scr[3:4, :]
    area_p = pp_scr[4:5, :]

    tgt = tgt_ref[0]
    tx1 = tgt[:, 0:1]
    ty1 = tgt[:, 1:2]
    tx2 = tgt[:, 2:3]
    ty2 = tgt[:, 3:4]
    area_t = (tx2 - tx1) * (ty2 - ty1)

    iw = jnp.clip(jnp.minimum(px2, tx2) - jnp.maximum(px1, tx1), 0.0, None)
    ih = jnp.clip(jnp.minimum(py2, ty2) - jnp.maximum(py1, ty1), 0.0, None)
    inter = iw * ih
    union = area_t + area_p - inter
    iou = inter / union

    iota_t = it_scr[...]
    iota_p = lax.broadcasted_iota(jnp.int32, (1, P), 1)

    best_ov = jnp.max(iou, axis=0, keepdims=True)
    bti = jnp.argmax(iou, axis=0, keepdims=True)

    bpi = jnp.argmax(iou, axis=1, keepdims=True)
    bestmask = jnp.any(iota_p == bpi, axis=0, keepdims=True)

    pos = jnp.logical_or(best_ov >= _THRESHOLD, bestmask)
    posf = pos.astype(f32)
    num_pos = jnp.sum(posf)

    oh = (bti == iota_t).astype(f32)
    u0 = (tgt[:, 0:1] + tgt[:, 2:3]) / 2
    u1 = (tgt[:, 1:2] + tgt[:, 3:4]) / 2
    u2 = tgt[:, 2:3] - tgt[:, 0:1]
    u3 = tgt[:, 3:4] - tgt[:, 1:2]
    tgt_u = jnp.concatenate([u0, u1, u2, u3, tgt[:, 8:20]], axis=1)
    u = lax.dot_general(tgt_u, oh, (((0,), (0,)), ((), ())),
                        preferred_element_type=f32)

    _, v1 = _VARIANCE
    sub = lax.broadcasted_iota(jnp.int32, (16, P), 0)
    is_log = jnp.logical_or(sub == 2, sub == 3)
    center = cs_scr[0:16]
    rscale = cs_scr[16:32]
    w = (u - center) * rscale
    g = jnp.where(is_log, jnp.log(w) * (1.0 / v1), w)

    loc = loc_ref[0]
    d = loc - g
    ad = jnp.abs(d)
    sl1 = jnp.where(ad < 1.0, 0.5 * d * d, ad - 0.5)
    loss_l_img = jnp.sum(sl1 * posf)

    x = conf_ref[0]
    lca = jnp.maximum(x, 0.0) - x * posf + jnp.log1p(jnp.exp(-jnp.abs(x)))
    posbce = jnp.sum(lca * posf)

    m_scr[pl.ds(i, 1), :] = jnp.where(pos, 0.0, lca)
    lane = lax.broadcasted_iota(jnp.int32, (1, 8), 1)
    row = jnp.where(lane == 0, num_pos,
                    jnp.where(lane == 1, loss_l_img,
                              jnp.where(lane == 2, posbce, 0.0)))
    st_scr[pl.ds(i, 1), :] = row


def _final_step(out_ref, m_scr, st_scr):
    P = m_scr.shape[1]
    m = m_scr[...]
    bits = lax.bitcast_convert_type(m, jnp.int32)
    npos = st_scr[:, 0:1]
    k = jnp.minimum(_NEG_POS_RATIO * npos, float(P - 1))

    def body(i, prefix):
        bit = 30 - i
        trial = prefix | jnp.left_shift(jnp.int32(1), bit)
        cnt = jnp.sum((bits >= trial).astype(jnp.float32), axis=1,
                      keepdims=True)
        return jnp.where(cnt >= k, trial, prefix)

    prefix0 = jnp.zeros(npos.shape, jnp.int32)
    prefix = lax.fori_loop(0, 31, body, prefix0)
    tval = lax.bitcast_convert_type(prefix, jnp.float32)
    gt = (m > tval).astype(jnp.float32)
    cgt = jnp.sum(gt, axis=1, keepdims=True)
    sgt = jnp.sum(m * gt, axis=1, keepdims=True)
    negsum = jnp.where(k > 0.0, sgt + (k - cgt) * tval, 0.0)

    loss_l = jnp.sum(st_scr[:, 1:2])
    loss_c = jnp.sum(st_scr[:, 2:3] + negsum)
    n = jnp.sum(npos)
    lane = lax.broadcasted_iota(jnp.int32, (1, 2), 1)
    out_ref[...] = jnp.where(lane == 0, loss_l / n, loss_c / n)


def _make_body(num):
    def body(tgt_ref, pr_ref, loc_ref, conf_ref, out_ref, m_scr, st_scr,
             pp_scr, cs_scr, it_scr):
        i = pl.program_id(0)

        @pl.when(i == 0)
        def _():
            _prior_setup(pr_ref, pp_scr, cs_scr, it_scr)

        _image_step(i, tgt_ref, pr_ref, loc_ref, conf_ref, m_scr, st_scr,
                    pp_scr, cs_scr, it_scr)

        @pl.when(i == num - 1)
        def _():
            _final_step(out_ref, m_scr, st_scr)

    return body


def _build(num, P, T):
    return pl.pallas_call(
        _make_body(num),
        grid=(num,),
        in_specs=[
            pl.BlockSpec((1, T, 20), lambda i: (i, 0, 0)),
            pl.BlockSpec((4, P), lambda i: (0, 0)),
            pl.BlockSpec((1, 16, P), lambda i: (i, 0, 0)),
            pl.BlockSpec((1, 1, P), lambda i: (i, 0, 0)),
        ],
        out_specs=pl.BlockSpec((1, 2), lambda i: (0, 0)),
        out_shape=jax.ShapeDtypeStruct((1, 2), jnp.float32),
        scratch_shapes=[
            pltpu.VMEM((num, P), jnp.float32),
            pltpu.VMEM((num, 8), jnp.float32),
            pltpu.VMEM((8, P), jnp.float32),
            pltpu.VMEM((32, P), jnp.float32),
            pltpu.VMEM((T, P), jnp.int32),
        ],
    )


def kernel(loc_data, conf_data, targets, priors):
    num, P = loc_data.shape[0], loc_data.shape[1]
    T = targets.shape[1]
    loc_t = jnp.transpose(loc_data, (0, 2, 1))
    conf = jnp.transpose(conf_data, (0, 2, 1))
    pr_t = priors[:P, :].T
    out = _build(num, P, T)(targets, pr_t, loc_t, conf)
    return out[0, 0], out[0, 1]


# speedup vs baseline: 1.0607x; 1.0034x over previous
import jax
import jax.numpy as jnp
from jax import lax
from jax.experimental import pallas as pl
from jax.experimental.pallas import tpu as pltpu

_VARIANCE = (0.1, 0.2)
_THRESHOLD = 0.35
_NEG_POS_RATIO = 3


def _prior_setup(pr_ref, pp_scr, cs_scr):
    P = pr_ref.shape[1]
    v0, _ = _VARIANCE
    pcx = pr_ref[0:1, :]
    pcy = pr_ref[1:2, :]
    pw = pr_ref[2:3, :]
    ph = pr_ref[3:4, :]
    px1 = pcx - pw / 2
    py1 = pcy - ph / 2
    px2 = pcx + pw / 2
    py2 = pcy + ph / 2
    area_p = (px2 - px1) * (py2 - py1)
    lane8 = lax.broadcasted_iota(jnp.int32, (8, P), 0)
    pp = jnp.where(lane8 == 0, jnp.broadcast_to(px1, (8, P)),
                   jnp.where(lane8 == 1, jnp.broadcast_to(py1, (8, P)),
                             jnp.where(lane8 == 2, jnp.broadcast_to(px2, (8, P)),
                                       jnp.where(lane8 == 3, jnp.broadcast_to(py2, (8, P)),
                                                 jnp.broadcast_to(area_p, (8, P))))))
    pp_scr[...] = pp
    sub = lax.broadcasted_iota(jnp.int32, (16, P), 0)
    is_y = (sub & 1) == 1
    is_log = jnp.logical_or(sub == 2, sub == 3)
    pcx16 = jnp.broadcast_to(pcx, (16, P))
    pcy16 = jnp.broadcast_to(pcy, (16, P))
    pw16 = jnp.broadcast_to(pw, (16, P))
    ph16 = jnp.broadcast_to(ph, (16, P))
    center = jnp.where(is_log, 0.0, jnp.where(is_y, pcy16, pcx16))
    scale = jnp.where(is_log, 1.0, v0) * jnp.where(is_y, ph16, pw16)
    cs_scr[0:16] = center
    cs_scr[16:32] = 1.0 / scale


def _image_step(i, tgt_ref, pr_ref, loc_ref, conf_ref, m_scr, st_scr,
                pp_scr, cs_scr):
    T = tgt_ref.shape[1]
    P = pr_ref.shape[1]
    f32 = jnp.float32

    px1 = pp_scr[0:1, :]
    py1 = pp_scr[1:2, :]
    px2 = pp_scr[2:3, :]
    py2 = pp_scr[3:4, :]
    area_p = pp_scr[4:5, :]

    tgt = tgt_ref[0]
    tx1 = tgt[:, 0:1]
    ty1 = tgt[:, 1:2]
    tx2 = tgt[:, 2:3]
    ty2 = tgt[:, 3:4]
    area_t = (tx2 - tx1) * (ty2 - ty1)

    iw = jnp.clip(jnp.minimum(px2, tx2) - jnp.maximum(px1, tx1), 0.0, None)
    ih = jnp.clip(jnp.minimum(py2, ty2) - jnp.maximum(py1, ty1), 0.0, None)
    inter = iw * ih
    union = area_t + area_p - inter
    iou = inter / union

    iota_t = lax.broadcasted_iota(jnp.int32, (T, P), 0)
    iota_p = lax.broadcasted_iota(jnp.int32, (1, P), 1)

    best_ov = jnp.max(iou, axis=0, keepdims=True)
    bti = jnp.argmax(iou, axis=0, keepdims=True)

    bpi = jnp.argmax(iou, axis=1, keepdims=True)
    bestmask = jnp.any(iota_p == bpi, axis=0, keepdims=True)

    pos = jnp.logical_or(best_ov >= _THRESHOLD, bestmask)
    posf = pos.astype(f32)
    num_pos = jnp.sum(posf)

    oh = (bti == iota_t).astype(f32)
    u0 = (tgt[:, 0:1] + tgt[:, 2:3]) / 2
    u1 = (tgt[:, 1:2] + tgt[:, 3:4]) / 2
    u2 = tgt[:, 2:3] - tgt[:, 0:1]
    u3 = tgt[:, 3:4] - tgt[:, 1:2]
    tgt_u = jnp.concatenate([u0, u1, u2, u3, tgt[:, 8:20]], axis=1)
    u = lax.dot_general(tgt_u, oh, (((0,), (0,)), ((), ())),
                        preferred_element_type=f32)

    _, v1 = _VARIANCE
    sub = lax.broadcasted_iota(jnp.int32, (16, P), 0)
    is_log = jnp.logical_or(sub == 2, sub == 3)
    center = cs_scr[0:16]
    rscale = cs_scr[16:32]
    w = (u - center) * rscale
    g = jnp.where(is_log, jnp.log(w) * (1.0 / v1), w)

    loc = loc_ref[0]
    d = loc - g
    ad = jnp.abs(d)
    sl1 = jnp.where(ad < 1.0, 0.5 * d * d, ad - 0.5)
    loss_l_img = jnp.sum(sl1 * posf)

    x = conf_ref[0]
    lca = jnp.maximum(x, 0.0) - x * posf + jnp.log1p(jnp.exp(-jnp.abs(x)))
    posbce = jnp.sum(lca * posf)

    m_scr[pl.ds(i, 1), :] = jnp.where(pos, 0.0, lca)
    lane = lax.broadcasted_iota(jnp.int32, (1, 8), 1)
    row = jnp.where(lane == 0, num_pos,
                    jnp.where(lane == 1, loss_l_img,
                              jnp.where(lane == 2, posbce, 0.0)))
    st_scr[pl.ds(i, 1), :] = row


def _final_step(out_ref, m_scr, st_scr):
    P = m_scr.shape[1]
    m = m_scr[...]
    bits = lax.bitcast_convert_type(m, jnp.int32)
    npos = st_scr[:, 0:1]
    k = jnp.minimum(_NEG_POS_RATIO * npos, float(P - 1))

    def body(i, prefix):
        bit = 30 - i
        trial = prefix | jnp.left_shift(jnp.int32(1), bit)
        cnt = jnp.sum((bits >= trial).astype(jnp.float32), axis=1,
                      keepdims=True)
        return jnp.where(cnt >= k, trial, prefix)

    prefix0 = jnp.zeros(npos.shape, jnp.int32)
    prefix = lax.fori_loop(0, 31, body, prefix0)
    tval = lax.bitcast_convert_type(prefix, jnp.float32)
    gt = (m > tval).astype(jnp.float32)
    cgt = jnp.sum(gt, axis=1, keepdims=True)
    sgt = jnp.sum(m * gt, axis=1, keepdims=True)
    negsum = jnp.where(k > 0.0, sgt + (k - cgt) * tval, 0.0)

    loss_l = jnp.sum(st_scr[:, 1:2])
    loss_c = jnp.sum(st_scr[:, 2:3] + negsum)
    n = jnp.sum(npos)
    lane = lax.broadcasted_iota(jnp.int32, (1, 2), 1)
    out_ref[...] = jnp.where(lane == 0, loss_l / n, loss_c / n)


def _make_body(num):
    def body(tgt_ref, pr_ref, loc_ref, conf_ref, out_ref, m_scr, st_scr,
             pp_scr, cs_scr):
        i = pl.program_id(0)

        @pl.when(i == 0)
        def _():
            _prior_setup(pr_ref, pp_scr, cs_scr)

        _image_step(i, tgt_ref, pr_ref, loc_ref, conf_ref, m_scr, st_scr,
                    pp_scr, cs_scr)

        @pl.when(i == num - 1)
        def _():
            _final_step(out_ref, m_scr, st_scr)

    return body


def _build(num, P, T):
    return pl.pallas_call(
        _make_body(num),
        grid=(num,),
        in_specs=[
            pl.BlockSpec((1, T, 20), lambda i: (i, 0, 0)),
            pl.BlockSpec((4, P), lambda i: (0, 0)),
            pl.BlockSpec((1, 16, P), lambda i: (i, 0, 0)),
            pl.BlockSpec((1, 1, P), lambda i: (i, 0, 0)),
        ],
        out_specs=pl.BlockSpec((1, 2), lambda i: (0, 0)),
        out_shape=jax.ShapeDtypeStruct((1, 2), jnp.float32),
        scratch_shapes=[
            pltpu.VMEM((num, P), jnp.float32),
            pltpu.VMEM((num, 8), jnp.float32),
            pltpu.VMEM((8, P), jnp.float32),
            pltpu.VMEM((32, P), jnp.float32),
        ],
    )


def kernel(loc_data, conf_data, targets, priors):
    num, P = loc_data.shape[0], loc_data.shape[1]
    T = targets.shape[1]
    loc_t = jnp.transpose(loc_data, (0, 2, 1))
    conf = jnp.transpose(conf_data, (0, 2, 1))
    pr_t = priors[:P, :].T
    out = _build(num, P, T)(targets, pr_t, loc_t, conf)
    return out[0, 0], out[0, 1]
